# baseline (speedup 1.0000x reference)
import functools

import jax
import jax.numpy as jnp
from jax import lax
from jax.experimental import pallas as pl
from jax.experimental.pallas import tpu as pltpu
from jax.experimental.pallas import tpu_sc as plsc

D = 128
B = 4096 * 200
CHUNK = 128

_info = plsc.get_sparse_core_info()
NC, NS = _info.num_cores, _info.num_subcores
NW = NC * NS
B_PER_W = B // NW
N_CHUNKS = B_PER_W // CHUNK

_mesh = plsc.VectorSubcoreMesh(core_axis_name="c", subcore_axis_name="s")


@functools.partial(
    pl.kernel,
    mesh=_mesh,
    out_type=jax.ShapeDtypeStruct((B, D), jnp.float32),
    scratch_types=[
        pltpu.VMEM((CHUNK,), jnp.int32),
        pltpu.VMEM((CHUNK, D), jnp.float32),
        pltpu.SemaphoreType.DMA,
    ],
)
def _emb_lookup(x_hbm, table_hbm, out_hbm, idx_v, rows_v, sem):
    wid = lax.axis_index("s") * NC + lax.axis_index("c")
    base = wid * B_PER_W

    def body(i, _):
        off = base + i * CHUNK
        pltpu.sync_copy(x_hbm.at[pl.ds(off, CHUNK)], idx_v)
        pltpu.async_copy(table_hbm.at[idx_v], rows_v, sem).wait()
        pltpu.sync_copy(rows_v, out_hbm.at[pl.ds(off, CHUNK)])
        return 0

    lax.fori_loop(0, N_CHUNKS, body, 0)


def kernel(x, table):
    x_flat = x.reshape(-1).astype(jnp.int32)
    out = _emb_lookup(x_flat, table)
    return out.reshape(x.shape + (D,))


# speedup vs baseline: 1.0035x; 1.0035x over previous
import functools

import jax
import jax.numpy as jnp
from jax import lax
from jax.experimental import pallas as pl
from jax.experimental.pallas import tpu as pltpu
from jax.experimental.pallas import tpu_sc as plsc

D = 128
B = 4096 * 200
CHUNK = 128
K = 2

_info = plsc.get_sparse_core_info()
NC, NS = _info.num_cores, _info.num_subcores
NW = NC * NS
B_PER_W = B // NW
N_CHUNKS = B_PER_W // CHUNK
N_GROUPS = N_CHUNKS // K

_mesh = plsc.VectorSubcoreMesh(core_axis_name="c", subcore_axis_name="s")


@functools.partial(
    pl.kernel,
    mesh=_mesh,
    out_type=jax.ShapeDtypeStruct((B, D), jnp.float32),
    scratch_types=[
        pltpu.VMEM((N_CHUNKS, CHUNK), jnp.int32),
        pltpu.VMEM((2 * K, CHUNK, D), jnp.float32),
        pltpu.SemaphoreType.DMA,
        pltpu.SemaphoreType.DMA,
    ],
)
def _emb_lookup(x_hbm, table_hbm, out_hbm, idx_all, rows, gsem, osem):
    wid = lax.axis_index("s") * NC + lax.axis_index("c")
    row_base = wid * B_PER_W
    chunk_base = wid * N_CHUNKS

    pltpu.sync_copy(x_hbm.at[pl.ds(chunk_base, N_CHUNKS)], idx_all)

    def drain_out():
        pltpu.make_async_copy(
            rows.at[0], out_hbm.at[pl.ds(row_base, CHUNK)], osem).wait()

    def do_group(g, half):
        copies = []
        for k in range(K):
            j = g * K + k
            copies.append(pltpu.async_copy(
                table_hbm.at[idx_all.at[j]], rows.at[half * K + k], gsem))
        for k in range(K):
            copies[k].wait()
        for k in range(K):
            j = g * K + k
            pltpu.make_async_copy(
                rows.at[half * K + k],
                out_hbm.at[pl.ds(row_base + j * CHUNK, CHUNK)],
                osem).start()

    def body(s, _):
        ge = 2 * s
        go = 2 * s + 1

        @pl.when(s >= 1)
        def _():
            drain_out()
            drain_out()

        do_group(ge, 0)

        @pl.when(s >= 1)
        def _():
            drain_out()
            drain_out()

        do_group(go, 1)
        return 0

    lax.fori_loop(0, N_GROUPS // 2, body, 0)

    for _ in range(2 * K):
        drain_out()


def kernel(x, table):
    x2d = x.reshape(-1, CHUNK).astype(jnp.int32)
    out = _emb_lookup(x2d, table)
    return out.reshape(x.shape + (D,))


# speedup vs baseline: 8.3579x; 8.3291x over previous
import functools

import jax
import jax.numpy as jnp
from jax import lax
from jax.experimental import pallas as pl
from jax.experimental.pallas import tpu as pltpu
from jax.experimental.pallas import tpu_sc as plsc

D = 128
B = 4096 * 200
CHUNK = 128
K = 2

_info = plsc.get_sparse_core_info()
NC, NS = _info.num_cores, _info.num_subcores
NW = NC * NS
B_PER_W = B // NW
N_CHUNKS = B_PER_W // CHUNK
N_GROUPS = N_CHUNKS // K

_mesh = plsc.VectorSubcoreMesh(core_axis_name="c", subcore_axis_name="s")


@functools.partial(
    pl.kernel,
    mesh=_mesh,
    out_type=jax.ShapeDtypeStruct((B, D), jnp.float32),
    scratch_types=[
        pltpu.VMEM((N_CHUNKS, CHUNK), jnp.int32),
        pltpu.VMEM((2 * K, CHUNK, D), jnp.float32),
        pltpu.SemaphoreType.DMA,
        pltpu.SemaphoreType.DMA,
    ],
)
def _emb_lookup(x_hbm, table_hbm, out_hbm, idx_all, rows, gsem, osem):
    wid = lax.axis_index("s") * NC + lax.axis_index("c")
    row_base = wid * B_PER_W
    chunk_base = wid * N_CHUNKS

    pltpu.sync_copy(x_hbm.at[pl.ds(chunk_base, N_CHUNKS)], idx_all)

    voff = jnp.full((16,), 3 * wid, jnp.int32)

    def rebase(i, _):
        r, c = i // 8, (i % 8) * 16
        idx_all[r, pl.ds(c, 16)] = idx_all[r, pl.ds(c, 16)] + voff
        return 0

    lax.fori_loop(0, N_CHUNKS * 8, rebase, 0)

    def drain_out():
        pltpu.make_async_copy(
            rows.at[0], out_hbm.at[pl.ds(row_base, CHUNK)], osem).wait()

    def do_group(g, half):
        copies = []
        for k in range(K):
            j = g * K + k
            copies.append(pltpu.async_copy(
                table_hbm.at[idx_all.at[j]], rows.at[half * K + k], gsem))
        for k in range(K):
            copies[k].wait()
        for k in range(K):
            j = g * K + k
            pltpu.make_async_copy(
                rows.at[half * K + k],
                out_hbm.at[pl.ds(row_base + j * CHUNK, CHUNK)],
                osem).start()

    def body(s, _):
        ge = 2 * s
        go = 2 * s + 1

        @pl.when(s >= 1)
        def _():
            drain_out()
            drain_out()

        do_group(ge, 0)

        @pl.when(s >= 1)
        def _():
            drain_out()
            drain_out()

        do_group(go, 1)
        return 0

    lax.fori_loop(0, N_GROUPS // 2, body, 0)

    for _ in range(2 * K):
        drain_out()


def kernel(x, table):
    x2d = x.reshape(-1, CHUNK).astype(jnp.int32)
    table_rep = jnp.tile(table, (NW, 1))
    out = _emb_lookup(x2d, table_rep)
    return out.reshape(x.shape + (D,))


# speedup vs baseline: 12.1653x; 1.4555x over previous
import functools

import jax
import jax.numpy as jnp
from jax import lax
from jax.experimental import pallas as pl
from jax.experimental.pallas import tpu as pltpu
from jax.experimental.pallas import tpu_sc as plsc

D = 128
B = 4096 * 200
CHUNK = 128
SLOTS = 2
U = 4

_info = plsc.get_sparse_core_info()
NC, NS = _info.num_cores, _info.num_subcores
NW = NC * NS
B_PER_W = B // NW
N_CHUNKS = B_PER_W // CHUNK
N_GROUPS = N_CHUNKS // SLOTS

_mesh = plsc.VectorSubcoreMesh(core_axis_name="c", subcore_axis_name="s")


@functools.partial(
    pl.kernel,
    mesh=_mesh,
    out_type=jax.ShapeDtypeStruct((B, D), jnp.float32),
    scratch_types=[
        pltpu.VMEM((B_PER_W,), jnp.int32),
        pltpu.VMEM((3 * D,), jnp.float32),
        pltpu.VMEM((SLOTS, CHUNK, D), jnp.float32),
        pltpu.SemaphoreType.DMA,
    ],
    compiler_params=pltpu.CompilerParams(needs_layout_passes=False),
)
def _emb_lookup(x_hbm, table_hbm, out_hbm, idx_flat, table_v, rows, osem):
    wid = lax.axis_index("s") * NC + lax.axis_index("c")
    row_base = wid * B_PER_W

    pltpu.sync_copy(x_hbm.at[pl.ds(row_base, B_PER_W)], idx_flat)
    pltpu.sync_copy(table_hbm, table_v)

    def scale_body(i, _):
        v = idx_flat[pl.ds(i * 16, 16)]
        idx_flat[pl.ds(i * 16, 16)] = v * D
        return 0

    lax.fori_loop(0, B_PER_W // 16, scale_body, 0)

    iota = lax.broadcasted_iota(jnp.int32, (16,), 0)
    cols = [iota + 16 * p for p in range(D // 16)]

    def compute_chunk(j, slot):
        jbase = j * CHUNK

        def row_body(rr, _):
            for u in range(U):
                r = rr * U + u
                offv = plsc.load_gather(idx_flat, [jnp.full((16,), jbase + r)])
                for p in range(D // 16):
                    vals = plsc.load_gather(table_v, [offv + cols[p]])
                    rows[slot, r, pl.ds(16 * p, 16)] = vals
            return 0

        lax.fori_loop(0, CHUNK // U, row_body, 0)

    def drain_out():
        pltpu.make_async_copy(
            rows.at[0], out_hbm.at[pl.ds(row_base, CHUNK)], osem).wait()

    def body(g, _):
        for b in range(SLOTS):
            j = g * SLOTS + b

            @pl.when(g >= 1)
            def _():
                drain_out()

            compute_chunk(j, b)
            pltpu.make_async_copy(
                rows.at[b],
                out_hbm.at[pl.ds(row_base + j * CHUNK, CHUNK)],
                osem).start()
        return 0

    lax.fori_loop(0, N_GROUPS, body, 0)

    for _ in range(SLOTS):
        drain_out()


def kernel(x, table):
    x1d = x.reshape(-1).astype(jnp.int32)
    out = _emb_lookup(x1d, table.reshape(-1))
    return out.reshape(x.shape + (D,))


# speedup vs baseline: 14.3602x; 1.1804x over previous
import functools

import jax
import jax.numpy as jnp
from jax import lax
from jax.experimental import pallas as pl
from jax.experimental.pallas import tpu as pltpu
from jax.experimental.pallas import tpu_sc as plsc

D = 128
B = 4096 * 200
CHUNK = 128
SLOTS = 2
U = 4

_info = plsc.get_sparse_core_info()
NC, NS = _info.num_cores, _info.num_subcores
NW = NC * NS
B_PER_W = B // NW
N_CHUNKS = B_PER_W // CHUNK
N_GROUPS = N_CHUNKS // SLOTS

_mesh = plsc.VectorSubcoreMesh(core_axis_name="c", subcore_axis_name="s")


@functools.partial(
    pl.kernel,
    mesh=_mesh,
    out_type=jax.ShapeDtypeStruct((B, D), jnp.float32),
    scratch_types=[
        pltpu.VMEM((B_PER_W,), jnp.int32),
        pltpu.VMEM((3 * D,), jnp.float32),
        pltpu.VMEM((SLOTS, CHUNK, D), jnp.float32),
        pltpu.SemaphoreType.DMA,
    ],
    compiler_params=pltpu.CompilerParams(needs_layout_passes=False),
)
def _emb_lookup(x_hbm, table_hbm, out_hbm, idx_flat, table_v, rows, osem):
    wid = lax.axis_index("s") * NC + lax.axis_index("c")
    row_base = wid * B_PER_W

    pltpu.sync_copy(x_hbm.at[pl.ds(row_base, B_PER_W)], idx_flat)
    pltpu.sync_copy(table_hbm, table_v)

    def scale_body(i, _):
        v = idx_flat[pl.ds(i * 16, 16)]
        idx_flat[pl.ds(i * 16, 16)] = v * D
        return 0

    lax.fori_loop(0, B_PER_W // 16, scale_body, 0)

    def compute_chunk(j, slot):
        jbase = j * CHUNK

        def row_body(rr, _):
            offv = idx_flat[pl.ds(jbase + rr * 16, 16)]
            for u in range(16):
                s = offv[u]
                for p in range(D // 16):
                    rows[slot, rr * 16 + u, pl.ds(16 * p, 16)] = (
                        table_v[pl.ds(s + 16 * p, 16)])
            return 0

        lax.fori_loop(0, CHUNK // 16, row_body, 0)

    def drain_out():
        pltpu.make_async_copy(
            rows.at[0], out_hbm.at[pl.ds(row_base, CHUNK)], osem).wait()

    def body(g, _):
        for b in range(SLOTS):
            j = g * SLOTS + b

            @pl.when(g >= 1)
            def _():
                drain_out()

            compute_chunk(j, b)
            pltpu.make_async_copy(
                rows.at[b],
                out_hbm.at[pl.ds(row_base + j * CHUNK, CHUNK)],
                osem).start()
        return 0

    lax.fori_loop(0, N_GROUPS, body, 0)

    for _ in range(SLOTS):
        drain_out()


def kernel(x, table):
    x1d = x.reshape(-1).astype(jnp.int32)
    out = _emb_lookup(x1d, table.reshape(-1))
    return out.reshape(x.shape + (D,))


# speedup vs baseline: 54.2746x; 3.7795x over previous
import functools

import jax
import jax.numpy as jnp
from jax import lax
from jax.experimental import pallas as pl
from jax.experimental.pallas import tpu as pltpu
from jax.experimental.pallas import tpu_sc as plsc

D = 128
B = 4096 * 200
CHUNK = 128
SLOTS = 2

_info = plsc.get_sparse_core_info()
NC, NS = _info.num_cores, _info.num_subcores
NW = NC * NS
B_PER_W = B // NW
N_CHUNKS = B_PER_W // CHUNK
N_GROUPS = N_CHUNKS // SLOTS

_mesh = plsc.VectorSubcoreMesh(core_axis_name="c", subcore_axis_name="s")


@functools.partial(
    pl.kernel,
    mesh=_mesh,
    out_type=jax.ShapeDtypeStruct((B, D), jnp.float32),
    scratch_types=[
        pltpu.VMEM((N_CHUNKS, CHUNK), jnp.int32),
        pltpu.VMEM_SHARED((3, D), jnp.float32),
        pltpu.VMEM((SLOTS, CHUNK, D), jnp.float32),
        pltpu.SemaphoreType.DMA,
        pltpu.SemaphoreType.DMA,
    ],
)
def _emb_lookup(x_hbm, table_hbm, out_hbm, idx_all, table_v, rows, gsem,
                osem):
    wid = lax.axis_index("s") * NC + lax.axis_index("c")
    row_base = wid * B_PER_W
    chunk_base = wid * N_CHUNKS

    pltpu.sync_copy(x_hbm.at[pl.ds(chunk_base, N_CHUNKS)], idx_all)

    @pl.when(lax.axis_index("s") == 0)
    def _():
        pltpu.sync_copy(table_hbm, table_v)

    plsc.subcore_barrier()

    def drain_out():
        pltpu.make_async_copy(
            rows.at[0], out_hbm.at[pl.ds(row_base, CHUNK)], osem).wait()

    def body(g, _):
        copies = []
        for b in range(SLOTS):
            j = g * SLOTS + b

            @pl.when(g >= 1)
            def _():
                drain_out()

            copies.append(pltpu.async_copy(
                table_v.at[idx_all.at[j]], rows.at[b], gsem))
        for b in range(SLOTS):
            j = g * SLOTS + b
            copies[b].wait()
            pltpu.make_async_copy(
                rows.at[b],
                out_hbm.at[pl.ds(row_base + j * CHUNK, CHUNK)],
                osem).start()
        return 0

    lax.fori_loop(0, N_GROUPS, body, 0)

    for _ in range(SLOTS):
        drain_out()


def kernel(x, table):
    x2d = x.reshape(-1, CHUNK).astype(jnp.int32)
    out = _emb_lookup(x2d, table)
    return out.reshape(x.shape + (D,))


# speedup vs baseline: 55.1057x; 1.0153x over previous
import functools

import jax
import jax.numpy as jnp
from jax import lax
from jax.experimental import pallas as pl
from jax.experimental.pallas import tpu as pltpu
from jax.experimental.pallas import tpu_sc as plsc

D = 128
B = 4096 * 200
CHUNK = 128
SLOTS = 4

_info = plsc.get_sparse_core_info()
NC, NS = _info.num_cores, _info.num_subcores
NW = NC * NS
B_PER_W = B // NW
N_CHUNKS = B_PER_W // CHUNK
N_GROUPS = N_CHUNKS // SLOTS

_mesh = plsc.VectorSubcoreMesh(core_axis_name="c", subcore_axis_name="s")


@functools.partial(
    pl.kernel,
    mesh=_mesh,
    out_type=jax.ShapeDtypeStruct((B, D), jnp.float32),
    scratch_types=[
        pltpu.VMEM((N_CHUNKS, CHUNK), jnp.int32),
        pltpu.VMEM_SHARED((3, D), jnp.float32),
        pltpu.VMEM((SLOTS, CHUNK, D), jnp.float32),
        pltpu.SemaphoreType.DMA,
        pltpu.SemaphoreType.DMA,
    ],
)
def _emb_lookup(x_hbm, table_hbm, out_hbm, idx_all, table_v, rows, gsem,
                osem):
    wid = lax.axis_index("s") * NC + lax.axis_index("c")
    row_base = wid * B_PER_W
    chunk_base = wid * N_CHUNKS

    pltpu.sync_copy(x_hbm.at[pl.ds(chunk_base, N_CHUNKS)], idx_all)

    @pl.when(lax.axis_index("s") == 0)
    def _():
        pltpu.sync_copy(table_hbm, table_v)

    plsc.subcore_barrier()

    def drain_out():
        pltpu.make_async_copy(
            rows.at[0], out_hbm.at[pl.ds(row_base, CHUNK)], osem).wait()

    def body(g, _):
        copies = []
        for b in range(SLOTS):
            j = g * SLOTS + b

            @pl.when(g >= 1)
            def _():
                drain_out()

            copies.append(pltpu.async_copy(
                table_v.at[idx_all.at[j]], rows.at[b], gsem))
        for b in range(SLOTS):
            j = g * SLOTS + b
            copies[b].wait()
            pltpu.make_async_copy(
                rows.at[b],
                out_hbm.at[pl.ds(row_base + j * CHUNK, CHUNK)],
                osem).start()
        return 0

    lax.fori_loop(0, N_GROUPS, body, 0)

    for _ in range(SLOTS):
        drain_out()


def kernel(x, table):
    x2d = x.reshape(-1, CHUNK).astype(jnp.int32)
    out = _emb_lookup(x2d, table)
    return out.reshape(x.shape + (D,))
